# baseline (speedup 1.0000x reference)
import jax
import jax.numpy as jnp
from jax.experimental import pallas as pl
from jax.experimental.pallas import tpu as pltpu

L = 15
SCALE_BOUND = 0.11
LIKELIHOOD_BOUND = 1e-09
_INV_SQRT2 = 0.7071067811865476


def _tc_body(w2_ref, nbb_ref, x_ref, s_ref, m_ref, out_ref, lik_ref):
    x = x_ref[...]
    bx = x * w2_ref[L]
    acc = w2_ref[0] * jnp.tanh(bx + nbb_ref[0])
    for i in range(1, L):
        acc = acc + w2_ref[i] * jnp.tanh(bx + nbb_ref[i])
    out_ref[...] = acc + m_ref[...]
    sb = jnp.maximum(s_ref[...], SCALE_BOUND)
    rk = _INV_SQRT2 / sb
    zu = (0.5 - acc) * rk
    zl = (-0.5 - acc) * rk
    lik = 0.5 * (jax.lax.erf(zu) - jax.lax.erf(zl))
    lik_ref[...] = jnp.maximum(lik, LIKELIHOOD_BOUND)


def kernel(inputs, scales, means, w, b, beta):
    shape = inputs.shape
    n = inputs.size
    cols = 1024
    rows = n // cols
    x2 = inputs.reshape(rows, cols)
    s2 = scales.reshape(rows, cols)
    m2 = means.reshape(rows, cols)

    w2 = jnp.concatenate([w * 0.5, beta.reshape(1)]).astype(jnp.float32)
    nbb = (-beta * b).astype(jnp.float32)

    block_rows = 128
    grid = (rows // block_rows,)
    out2, lik2 = pl.pallas_call(
        _tc_body,
        grid=grid,
        in_specs=[
            pl.BlockSpec(memory_space=pltpu.SMEM),
            pl.BlockSpec(memory_space=pltpu.SMEM),
            pl.BlockSpec((block_rows, cols), lambda i: (i, 0)),
            pl.BlockSpec((block_rows, cols), lambda i: (i, 0)),
            pl.BlockSpec((block_rows, cols), lambda i: (i, 0)),
        ],
        out_specs=[
            pl.BlockSpec((block_rows, cols), lambda i: (i, 0)),
            pl.BlockSpec((block_rows, cols), lambda i: (i, 0)),
        ],
        out_shape=[
            jax.ShapeDtypeStruct((rows, cols), jnp.float32),
            jax.ShapeDtypeStruct((rows, cols), jnp.float32),
        ],
    )(w2, nbb, x2, s2, m2)
    return out2.reshape(shape), lik2.reshape(shape)
